# Initial kernel scaffold; baseline (speedup 1.0000x reference)
import jax
import jax.numpy as jnp
from jax.experimental import pallas as pl


def kernel(x, Wr, br, We, be):
    raise NotImplementedError("write your pallas kernel here")


# speedup vs baseline: 3.0949x; 3.0949x over previous
import functools

import jax
import jax.numpy as jnp
from jax import lax
from jax.experimental import pallas as pl
from jax.experimental.pallas import tpu as pltpu
from jax.experimental.pallas import tpu_sc as plsc

N_TOK = 16384
N_EXP = 64
D = 768
BM = 64
MP = N_TOK + N_EXP * BM
NT = MP // BM
TRASH = N_TOK
OUT_ROWS = N_TOK + 8
TB = 1024
NSUB = 16
TPW = N_TOK // NSUB
CPW = MP // NSUB
GPW = TPW // 16


def _router_body(x_ref, wr_ref, br_ref, routes_ref, rpm_ref, rps_ref, cnt_ref):
    i = pl.program_id(0)
    x = x_ref[...]
    wr = wr_ref[...]
    logits = lax.dot_general(x, wr, (((1,), (1,)), ((), ())),
                             preferred_element_type=jnp.float32)
    logits = logits + br_ref[...]
    prob = jax.nn.softmax(logits, axis=-1)
    rpm = jnp.max(prob, axis=-1)
    eiota = lax.broadcasted_iota(jnp.int32, (TB, N_EXP), 1)
    routes = jnp.min(jnp.where(prob == rpm[:, None], eiota, N_EXP), axis=-1)
    onehot = (eiota == routes[:, None]).astype(jnp.float32)
    routes_ref[...] = routes.reshape(TB // 128, 128)
    rpm_ref[...] = rpm.reshape(TB // 128, 128)

    @pl.when(i == 0)
    def _():
        rps_ref[...] = jnp.zeros_like(rps_ref)
        cnt_ref[...] = jnp.zeros_like(cnt_ref)

    rps_ref[...] += jnp.sum(prob, axis=0).reshape(1, N_EXP)
    cnt_ref[...] += jnp.sum(onehot, axis=0).reshape(1, N_EXP)


def _router(x, Wr, br):
    n_blk = N_TOK // TB
    return pl.pallas_call(
        _router_body,
        grid=(n_blk,),
        in_specs=[
            pl.BlockSpec((TB, D), lambda i: (i, 0)),
            pl.BlockSpec((N_EXP, D), lambda i: (0, 0)),
            pl.BlockSpec((1, N_EXP), lambda i: (0, 0)),
        ],
        out_specs=[
            pl.BlockSpec((TB // 128, 128), lambda i: (i, 0)),
            pl.BlockSpec((TB // 128, 128), lambda i: (i, 0)),
            pl.BlockSpec((1, N_EXP), lambda i: (0, 0)),
            pl.BlockSpec((1, N_EXP), lambda i: (0, 0)),
        ],
        out_shape=[
            jax.ShapeDtypeStruct((N_TOK // 128, 128), jnp.int32),
            jax.ShapeDtypeStruct((N_TOK // 128, 128), jnp.float32),
            jax.ShapeDtypeStruct((1, N_EXP), jnp.float32),
            jax.ShapeDtypeStruct((1, N_EXP), jnp.float32),
        ],
    )(x, Wr, br.reshape(1, N_EXP))


def _bin_kernel_body(routes_hbm, rpm2_hbm, base_hbm,
                     gid_hbm, dest_hbm, scal_hbm,
                     routes_v, hist_v, cnt_v, allh_v,
                     base_v, zero_v, trash_v, slots_v, gvals_v, rpm2_v,
                     hist_s, cnt_s, hist_sh, sem):
    s = lax.axis_index("s")
    tok0 = s * TPW
    cb = s * CPW
    iot = lax.iota(jnp.int32, 16)

    for k in range(8):
        zero_v[pl.ds(16 * k, 16)] = jnp.zeros((16,), jnp.int32)
        trash_v[pl.ds(16 * k, 16)] = jnp.full((16,), TRASH, jnp.int32)

    @pl.loop(0, CPW // 128)
    def _(j):
        pltpu.sync_copy(zero_v, gid_hbm.at[pl.ds(cb + j * 128, 128)])
        pltpu.sync_copy(trash_v, dest_hbm.at[pl.ds(cb + j * 128, 128)])

    pltpu.sync_copy(routes_hbm.at[pl.ds(tok0, TPW)], routes_v)
    pltpu.sync_copy(rpm2_hbm.at[s], rpm2_v)
    pltpu.sync_copy(base_hbm, base_v)

    for l in range(N_EXP):
        hist_s[l] = 0

    @pl.loop(0, TPW // 16)
    def _(g):
        ev = routes_v[pl.ds(g * 16, 16)]
        for l in range(16):
            e = ev[l]
            hist_s[e] = hist_s[e] + 1

    for j in range(4):
        acc = jnp.zeros((16,), jnp.int32)
        for l in range(16):
            hs = hist_s[j * 16 + l]
            acc = jnp.where(iot == l, lax.broadcast(hs, (16,)), acc)
        hist_v[pl.ds(j * 16, 16)] = acc

    pltpu.sync_copy(hist_v, hist_sh.at[pl.ds(s * N_EXP, N_EXP)])
    plsc.subcore_barrier()
    pltpu.sync_copy(hist_sh, allh_v)

    for j in range(4):
        pre = jnp.zeros((16,), jnp.int32)
        for sp in range(NSUB - 1):
            h = allh_v[pl.ds(sp * N_EXP + j * 16, 16)]
            m = lax.broadcast((s > sp).astype(jnp.int32), (16,))
            pre = pre + h * m
        cv = base_v[pl.ds(j * 16, 16)] + pre
        cnt_v[pl.ds(j * 16, 16)] = cv
        for l in range(16):
            cnt_s[j * 16 + l] = cv[l]

    for r in range(8):
        @pl.loop(0, 8)
        def _(g2, r=r):
            g = r * 8 + g2
            ev = routes_v[pl.ds(g * 16, 16)]
            slot_acc = jnp.zeros((16,), jnp.int32)
            for l in range(16):
                e = ev[l]
                sl = cnt_s[e]
                cnt_s[e] = sl + 1
                slot_acc = jnp.where(iot == l, lax.broadcast(sl, (16,)),
                                     slot_acc)
            c0 = g2 * 16
            slots_v[r, pl.ds(c0, 16)] = slot_acc
            gvals_v[r, pl.ds(c0, 16)] = iot + (tok0 + g * 16)

    for r in range(8):
        pltpu.async_copy(gvals_v.at[r], gid_hbm.at[slots_v.at[r]], sem).wait()
        pltpu.async_copy(gvals_v.at[r], dest_hbm.at[slots_v.at[r]], sem).wait()
        pltpu.async_copy(rpm2_v.at[r], scal_hbm.at[slots_v.at[r]], sem).wait()


def _binning(routes, rpm, base):
    mesh = plsc.VectorSubcoreMesh(core_axis_name="c", subcore_axis_name="s",
                                  num_cores=1)
    f = pl.kernel(
        _bin_kernel_body,
        out_type=[
            jax.ShapeDtypeStruct((MP,), jnp.int32),
            jax.ShapeDtypeStruct((MP,), jnp.int32),
            jax.ShapeDtypeStruct((MP,), jnp.float32),
        ],
        mesh=mesh,
        scratch_types=[
            pltpu.VMEM((TPW,), jnp.int32),
            pltpu.VMEM((N_EXP,), jnp.int32),
            pltpu.VMEM((N_EXP,), jnp.int32),
            pltpu.VMEM((NSUB * N_EXP,), jnp.int32),
            pltpu.VMEM((N_EXP,), jnp.int32),
            pltpu.VMEM((128,), jnp.int32),
            pltpu.VMEM((128,), jnp.int32),
            pltpu.VMEM((8, 128), jnp.int32),
            pltpu.VMEM((8, 128), jnp.int32),
            pltpu.VMEM((8, 128), jnp.float32),
            pltpu.SMEM((N_EXP,), jnp.int32),
            pltpu.SMEM((N_EXP,), jnp.int32),
            pltpu.VMEM_SHARED((NSUB * N_EXP,), jnp.int32),
            pltpu.SemaphoreType.DMA,
        ],
    )
    return f(routes, rpm.reshape(NSUB, 8, 128), base)


_SPW = MP // 32
_CH = 32
_NCH = _SPW // _CH


def _gather_body(x_hbm, gid2_hbm, xs_hbm, gid2_v, rows_v, sem):
    w = lax.axis_index("s") * 2 + lax.axis_index("c")
    pltpu.sync_copy(gid2_hbm.at[w], gid2_v)

    @pl.loop(0, _NCH)
    def _(c):
        pltpu.async_copy(x_hbm.at[gid2_v.at[c]], rows_v, sem).wait()
        pltpu.sync_copy(rows_v, xs_hbm.at[pl.ds(w * _SPW + c * _CH, _CH)])


def _gather(x, gid):
    mesh = plsc.VectorSubcoreMesh(core_axis_name="c", subcore_axis_name="s")
    f = pl.kernel(
        _gather_body,
        out_type=jax.ShapeDtypeStruct((MP, D), jnp.float32),
        mesh=mesh,
        scratch_types=[
            pltpu.VMEM((_NCH, _CH), jnp.int32),
            pltpu.VMEM((_CH, D), jnp.float32),
            pltpu.SemaphoreType.DMA,
        ],
    )
    return f(x, gid.reshape(32, _NCH, _CH))


def _ffn_body(te_ref, xs_ref, we_ref, be_ref, sc_ref, ys_ref):
    w = we_ref[0]
    y = lax.dot_general(xs_ref[...], w, (((1,), (1,)), ((), ())),
                        preferred_element_type=jnp.float32)
    ys_ref[...] = jnp.maximum(y + be_ref[0], 0.0) * sc_ref[...]


def _ffn(xs, We, be, scal, te):
    grid_spec = pltpu.PrefetchScalarGridSpec(
        num_scalar_prefetch=1,
        grid=(NT,),
        in_specs=[
            pl.BlockSpec((BM, D), lambda i, te: (i, 0)),
            pl.BlockSpec((1, D, D), lambda i, te: (te[i], 0, 0)),
            pl.BlockSpec((1, 1, D), lambda i, te: (te[i], 0, 0)),
            pl.BlockSpec((BM, 1), lambda i, te: (i, 0)),
        ],
        out_specs=pl.BlockSpec((BM, D), lambda i, te: (i, 0)),
    )
    return pl.pallas_call(
        _ffn_body,
        grid_spec=grid_spec,
        out_shape=jax.ShapeDtypeStruct((MP, D), jnp.float32),
    )(te, xs, We, be.reshape(N_EXP, 1, D), scal.reshape(MP, 1))


def _scatter_body(ys_hbm, did2_hbm, out_hbm, did2_v, rows_v, sem):
    w = lax.axis_index("s") * 2 + lax.axis_index("c")
    pltpu.sync_copy(did2_hbm.at[w], did2_v)

    @pl.loop(0, _NCH)
    def _(c):
        pltpu.sync_copy(ys_hbm.at[pl.ds(w * _SPW + c * _CH, _CH)], rows_v)
        pltpu.async_copy(rows_v, out_hbm.at[did2_v.at[c]], sem).wait()


def _scatter(ys, dest):
    mesh = plsc.VectorSubcoreMesh(core_axis_name="c", subcore_axis_name="s")
    f = pl.kernel(
        _scatter_body,
        out_type=jax.ShapeDtypeStruct((OUT_ROWS, D), jnp.float32),
        mesh=mesh,
        scratch_types=[
            pltpu.VMEM((_NCH, _CH), jnp.int32),
            pltpu.VMEM((_CH, D), jnp.float32),
            pltpu.SemaphoreType.DMA,
        ],
    )
    return f(ys, dest.reshape(32, _NCH, _CH))


def kernel(x, Wr, br, We, be):
    routes2, rpm2, rps2, cnt2 = _router(x, Wr, br)
    routes = routes2.reshape(N_TOK)
    rpm = rpm2.reshape(N_TOK)
    counts = cnt2.reshape(N_EXP)

    ci = counts.astype(jnp.int32)
    padded = (ci + (BM - 1)) // BM * BM
    ends = jnp.cumsum(padded)
    base = (ends - padded).astype(jnp.int32)
    te = jnp.searchsorted(ends, jnp.arange(NT, dtype=jnp.int32) * BM,
                          side="right").astype(jnp.int32)
    te = jnp.minimum(te, N_EXP - 1)

    gid, dest, scal = _binning(routes, rpm, base)
    xs = _gather(x, gid)
    ys = _ffn(xs, We, be, scal, te)
    outb = _scatter(ys, dest)
    return outb[:N_TOK], counts, rps2.reshape(N_EXP), rpm
